# Initial kernel scaffold; baseline (speedup 1.0000x reference)
import jax
import jax.numpy as jnp
from jax.experimental import pallas as pl


def kernel(pos, edge_index, atom_types, mol_ids, vdw_radii):
    raise NotImplementedError("write your pallas kernel here")


# speedup vs baseline: 513.5058x; 513.5058x over previous
import functools

import jax
import jax.numpy as jnp
import numpy as np
from jax import lax
from jax.experimental import pallas as pl
from jax.experimental.pallas import tpu as pltpu
from jax.experimental.pallas import tpu_sc as plsc

_TURNOVER = 10.0
_NUM_GRAPHS = 256
_LANES = 16
_NW = 32


def _rsqrt_nr(x):
    i = plsc.bitcast(x, jnp.int32)
    i = np.int32(0x5F3759DF) - (i >> 1)
    y = plsc.bitcast(i, jnp.float32)
    for _ in range(3):
        y = y * (jnp.float32(1.5) - jnp.float32(0.5) * x * y * y)
    return y


def _sc_body(n_nodes, e_per_w, posx_h, posy_h, posz_h, src_h, dst_h, at_h,
             mid_h, rad_h, zero_h, out_h,
             posx_v, posy_v, posz_v, src_v, dst_v, at_v, mid_v, rad_v, acc_v):
    c = lax.axis_index("c")
    s = lax.axis_index("s")
    wid = s * np.int32(2) + c
    base = wid * np.int32(e_per_w)

    pltpu.sync_copy(posx_h, posx_v)
    pltpu.sync_copy(posy_h, posy_v)
    pltpu.sync_copy(posz_h, posz_v)
    pltpu.sync_copy(at_h, at_v)
    pltpu.sync_copy(mid_h, mid_v)
    pltpu.sync_copy(rad_h, rad_v)
    pltpu.sync_copy(zero_h, acc_v)
    pltpu.sync_copy(src_h.at[pl.ds(base, e_per_w)], src_v)
    pltpu.sync_copy(dst_h.at[pl.ds(base, e_per_w)], dst_v)

    lane = lax.iota(jnp.int32, _LANES)
    j0 = jnp.zeros((_LANES,), jnp.int32)
    j1 = jnp.ones((_LANES,), jnp.int32)

    def _edge_vec(i):
        off = i * np.int32(_LANES)
        si = src_v[pl.ds(off, _LANES)]
        ti = dst_v[pl.ds(off, _LANES)]
        dx = plsc.load_gather(posx_v, [si]) - plsc.load_gather(posx_v, [ti])
        dy = plsc.load_gather(posy_v, [si]) - plsc.load_gather(posy_v, [ti])
        dz = plsc.load_gather(posz_v, [si]) - plsc.load_gather(posz_v, [ti])
        d2 = dx * dx + dy * dy + dz * dz + jnp.float32(1e-12)
        dc2 = jnp.maximum(d2, jnp.float32(0.0025))
        inv_d = _rsqrt_nr(dc2)
        a_s = plsc.load_gather(at_v, [si])
        a_t = plsc.load_gather(at_v, [ti])
        rv = plsc.load_gather(rad_v, [a_s]) + plsc.load_gather(rad_v, [a_t])
        q = rv * inv_d
        q2 = q * q
        r6 = q2 * q2 * q2
        lj = r6 * r6 - jnp.float32(2.0) * r6
        d = dc2 * inv_d
        ov = jnp.maximum(rv - d, jnp.float32(0.0))
        nov = ov / rv
        g = plsc.load_gather(mid_v, [ti])
        plsc.addupdate_scatter(acc_v, [j0, lane, g], lj)
        plsc.addupdate_scatter(acc_v, [j1, lane, g], nov)
        return i + np.int32(1)

    n_vec = np.int32(e_per_w // _LANES)
    lax.while_loop(lambda i: i < n_vec, _edge_vec, np.int32(0))

    row = wid * np.int32(_LANES)
    z = np.int32(0)
    o = np.int32(1)
    pltpu.sync_copy(acc_v.at[z], out_h.at[z, pl.ds(row, _LANES)])
    pltpu.sync_copy(acc_v.at[o], out_h.at[o, pl.ds(row, _LANES)])


def _tc_epilogue(p_ref, pot_ref, loss_ref, nov_ref):
    pot = jnp.sum(p_ref[0], axis=0, keepdims=True)
    nov = jnp.sum(p_ref[1], axis=0, keepdims=True)
    t = jnp.float32(_TURNOVER)
    safe = jnp.maximum(pot, t)
    loss = jnp.where(pot > t, t * (jnp.float32(1.0) + jnp.log(safe / t)), pot)
    pot_ref[...] = pot
    loss_ref[...] = loss
    nov_ref[...] = nov


def kernel(pos, edge_index, atom_types, mol_ids, vdw_radii):
    n_nodes = pos.shape[0]
    n_edges = edge_index.shape[1]
    e_per_w = n_edges // _NW

    pos32 = pos.astype(jnp.float32)
    posx = pos32[:, 0]
    posy = pos32[:, 1]
    posz = pos32[:, 2]
    src = edge_index[0].astype(jnp.int32)
    dst = edge_index[1].astype(jnp.int32)
    at = atom_types.astype(jnp.int32)
    mid = mol_ids.astype(jnp.int32)
    n_types = vdw_radii.shape[0]
    pad = (-n_types) % 128
    rad = jnp.pad(vdw_radii.astype(jnp.float32), (0, pad))
    zero = jnp.zeros((2, _LANES, _NUM_GRAPHS), jnp.float32)

    mesh = plsc.VectorSubcoreMesh(core_axis_name="c", subcore_axis_name="s")
    sc_call = pl.kernel(
        functools.partial(_sc_body, n_nodes, e_per_w),
        out_type=jax.ShapeDtypeStruct((2, _NW * _LANES, _NUM_GRAPHS),
                                      jnp.float32),
        mesh=mesh,
        scratch_types=[
            pltpu.VMEM((n_nodes,), jnp.float32),
            pltpu.VMEM((n_nodes,), jnp.float32),
            pltpu.VMEM((n_nodes,), jnp.float32),
            pltpu.VMEM((e_per_w,), jnp.int32),
            pltpu.VMEM((e_per_w,), jnp.int32),
            pltpu.VMEM((n_nodes,), jnp.int32),
            pltpu.VMEM((n_nodes,), jnp.int32),
            pltpu.VMEM((n_types + pad,), jnp.float32),
            pltpu.VMEM((2, _LANES, _NUM_GRAPHS), jnp.float32),
        ],
        compiler_params=pltpu.CompilerParams(needs_layout_passes=False),
    )
    partial_sums = sc_call(posx, posy, posz, src, dst, at, mid, rad, zero)

    pot, loss, nov = pl.pallas_call(
        _tc_epilogue,
        out_shape=(
            jax.ShapeDtypeStruct((1, _NUM_GRAPHS), jnp.float32),
            jax.ShapeDtypeStruct((1, _NUM_GRAPHS), jnp.float32),
            jax.ShapeDtypeStruct((1, _NUM_GRAPHS), jnp.float32),
        ),
    )(partial_sums)
    return (pot.reshape(_NUM_GRAPHS), loss.reshape(_NUM_GRAPHS),
            nov.reshape(_NUM_GRAPHS))
